# Initial kernel scaffold; baseline (speedup 1.0000x reference)
import jax
import jax.numpy as jnp
from jax.experimental import pallas as pl


def kernel(x, edge_index, W, b):
    raise NotImplementedError("write your pallas kernel here")


# speedup vs baseline: 7.6935x; 7.6935x over previous
import functools

import jax
import jax.numpy as jnp
from jax import lax
from jax.experimental import pallas as pl
from jax.experimental.pallas import tpu as pltpu
from jax.experimental.pallas import tpu_sc as plsc

NC = 2
NS = 16
NW = NC * NS
CHUNK = 128


def _sc_aggregate(n_rows, n_pad_rows, k_chunks, d):
    npad = n_rows + n_pad_rows
    ips = npad // NS

    mesh = plsc.VectorSubcoreMesh(core_axis_name="c", subcore_axis_name="s")

    @functools.partial(
        pl.kernel,
        out_type=jax.ShapeDtypeStruct((NC, npad, d), jnp.float32),
        mesh=mesh,
        scratch_types=[
            pltpu.VMEM((k_chunks, CHUNK), jnp.int32),
            pltpu.VMEM((k_chunks, CHUNK), jnp.int32),
            pltpu.VMEM((CHUNK, d), jnp.float32),
            pltpu.VMEM_SHARED((npad, d), jnp.float32),
        ],
    )
    def sc_kernel(x_hbm, src_hbm, dst_hbm, zeros_hbm, out_hbm,
                  src_v, dst_v, buf, acc):
        c = lax.axis_index("c")
        s = lax.axis_index("s")
        wid = s * NC + c

        pltpu.sync_copy(zeros_hbm, acc.at[pl.ds(s * ips, ips)])
        pltpu.sync_copy(src_hbm.at[wid], src_v)
        pltpu.sync_copy(dst_hbm.at[wid], dst_v)
        plsc.subcore_barrier()

        @pl.loop(0, k_chunks)
        def _(j):
            pltpu.sync_copy(x_hbm.at[src_v.at[j]], buf)
            pltpu.sync_copy(buf, acc.at[dst_v.at[j]], add=True)

        plsc.subcore_barrier()
        pltpu.sync_copy(acc.at[pl.ds(s * ips, ips)],
                        out_hbm.at[c, pl.ds(s * ips, ips)])

    return sc_kernel


def _combine_body(x_ref, p0_ref, p1_ref, w_ref, b_ref, o_ref):
    agg = p0_ref[0] + p1_ref[0]
    conv = lax.dot_general(
        agg, w_ref[...], (((1,), (0,)), ((), ())),
        precision=lax.Precision.HIGHEST,
        preferred_element_type=jnp.float32,
    )
    o_ref[...] = x_ref[...] + conv + b_ref[...]


@jax.jit
def kernel(x, edge_index, W, b):
    n, d = x.shape
    e = edge_index.shape[1]

    epw0 = e // NW
    k_chunks = -(-epw0 // CHUNK)
    k_chunks += k_chunks % 2
    epw = k_chunks * CHUNK
    pad_per_w = epw - epw0
    n_pad_rows = pad_per_w
    while (n + n_pad_rows) % NS:
        n_pad_rows += 1

    src = edge_index[0].reshape(NW, epw0)
    dst = edge_index[1].reshape(NW, epw0)
    pad_ar = jnp.arange(pad_per_w, dtype=jnp.int32)
    pad_src = jnp.broadcast_to((pad_ar * 997) % n, (NW, pad_per_w))
    pad_dst = jnp.broadcast_to(n + pad_ar % n_pad_rows, (NW, pad_per_w))
    src3 = jnp.concatenate([src, pad_src], axis=1).reshape(NW, k_chunks, CHUNK)
    dst3 = jnp.concatenate([dst, pad_dst], axis=1).reshape(NW, k_chunks, CHUNK)

    ips = (n + n_pad_rows) // NS
    zeros = jnp.zeros((ips, d), jnp.float32)

    partials = _sc_aggregate(n, n_pad_rows, k_chunks, d)(x, src3, dst3, zeros)

    blk = 1000
    nb = n // blk
    b2 = b.reshape(1, d)
    row_spec = pl.BlockSpec((blk, d), lambda i: (i, 0))
    out = pl.pallas_call(
        _combine_body,
        grid=(nb,),
        in_specs=[
            row_spec,
            pl.BlockSpec((1, blk, d), lambda i: (0, i, 0)),
            pl.BlockSpec((1, blk, d), lambda i: (1, i, 0)),
            pl.BlockSpec((d, d), lambda i: (0, 0)),
            pl.BlockSpec((1, d), lambda i: (0, 0)),
        ],
        out_specs=row_spec,
        out_shape=jax.ShapeDtypeStruct((n, d), jnp.float32),
    )(x, partials, partials, W, b2)
    return out


# speedup vs baseline: 12.8931x; 1.6758x over previous
import functools

import jax
import jax.numpy as jnp
from jax import lax
from jax.experimental import pallas as pl
from jax.experimental.pallas import tpu as pltpu
from jax.experimental.pallas import tpu_sc as plsc

NC = 2
NS = 16
NW = NC * NS
CHUNK = 128
NBUF = 3
IR = 6


def _sc_aggregate(n_rows, d, m_chunks):
    stripe = -(-n_rows // NS)
    stripe += (-stripe) % 8
    tail = n_rows - stripe * (NS - 1)
    assert tail > 0 and tail % 8 == 0 and stripe % 8 == 0
    base_c = m_chunks // NW
    extra = m_chunks - base_c * NW

    mesh = plsc.VectorSubcoreMesh(core_axis_name="c", subcore_axis_name="s")

    @functools.partial(
        pl.kernel,
        out_type=jax.ShapeDtypeStruct((NC, n_rows, d), jnp.float32),
        mesh=mesh,
        scratch_types=[
            pltpu.VMEM((IR, 1, CHUNK), jnp.int32),
            pltpu.VMEM((IR, 1, CHUNK), jnp.int32),
            pltpu.VMEM((CHUNK, d), jnp.float32),
            pltpu.VMEM((CHUNK, d), jnp.float32),
            pltpu.VMEM((CHUNK, d), jnp.float32),
            pltpu.SemaphoreType.DMA,
            pltpu.SemaphoreType.DMA,
            pltpu.SemaphoreType.DMA,
            pltpu.SemaphoreType.DMA,
            pltpu.SemaphoreType.DMA,
            pltpu.SemaphoreType.DMA,
            pltpu.SemaphoreType.DMA,
            pltpu.SemaphoreType.DMA,
            pltpu.SemaphoreType.DMA,
            pltpu.VMEM_SHARED((n_rows, d), jnp.float32),
        ],
    )
    def sc_kernel(x_hbm, src_hbm, dst_hbm, zeros_hbm, out_hbm,
                  src_r, dst_r, b0, b1, b2,
                  i0, i1, i2, i3, i4, i5, g0, g1, g2, acc):
        c = lax.axis_index("c")
        s = lax.axis_index("s")
        wid = s * NC + c
        bufs = (b0, b1, b2)
        gsems = (g0, g1, g2)
        isems = (i0, i1, i2, i3, i4, i5)

        start = base_c * wid + jnp.minimum(wid, extra)
        n_real = base_c + jnp.where(wid < extra, 1, 0)
        pad_hi = -(-(base_c + 1) // IR) * IR
        pad_lo = -(-base_c // IR) * IR
        t_loop = jnp.where(wid < extra, pad_hi, pad_lo)
        row0 = s * stripe

        @pl.when(s < NS - 1)
        def _():
            pltpu.sync_copy(zeros_hbm, acc.at[pl.ds(row0, stripe)])

        @pl.when(s == NS - 1)
        def _():
            pltpu.sync_copy(zeros_hbm.at[pl.ds(0, tail)],
                            acc.at[pl.ds((NS - 1) * stripe, tail)])

        def grow(t):
            return jnp.minimum(start + t, m_chunks - 1)

        def idx_load(t, slot):
            g = grow(t)
            pltpu.make_async_copy(src_hbm.at[g], src_r.at[slot],
                                  isems[slot]).start()
            pltpu.make_async_copy(dst_hbm.at[g], dst_r.at[slot],
                                  isems[slot]).start()

        def idx_wait(slot):
            pltpu.make_async_copy(src_hbm.at[0], src_r.at[slot],
                                  isems[slot]).wait()
            pltpu.make_async_copy(dst_hbm.at[0], dst_r.at[slot],
                                  isems[slot]).wait()

        def gather(islot, ring):
            return pltpu.make_async_copy(
                x_hbm.at[src_r.at[islot, 0]], bufs[ring], gsems[ring])

        for t in range(IR):
            idx_load(t, t)
        for t in range(NBUF):
            idx_wait(t)
            gather(t, t).start()
        plsc.subcore_barrier()

        @pl.loop(0, t_loop, step=IR)
        def _(t):
            for r in range(IR):
                tt = t + r
                ring = r % NBUF
                gather(r, ring).wait()

                @pl.when(tt < n_real)
                def _():
                    pltpu.sync_copy(bufs[ring], acc.at[dst_r.at[r, 0]],
                                    add=True)

                @pl.when(tt + IR < t_loop)
                def _():
                    idx_load(tt + IR, r)

                nslot = (r + NBUF) % IR

                @pl.when(tt + NBUF < t_loop)
                def _():
                    idx_wait(nslot)
                    gather(nslot, ring).start()

        plsc.subcore_barrier()
        @pl.when(s < NS - 1)
        def _():
            pltpu.sync_copy(acc.at[pl.ds(row0, stripe)],
                            out_hbm.at[c, pl.ds(row0, stripe)])

        @pl.when(s == NS - 1)
        def _():
            pltpu.sync_copy(acc.at[pl.ds((NS - 1) * stripe, tail)],
                            out_hbm.at[c, pl.ds((NS - 1) * stripe, tail)])

    return sc_kernel


def _combine_body(x_ref, p0_ref, p1_ref, w_ref, b_ref, o_ref):
    agg = p0_ref[0] + p1_ref[0]
    conv = lax.dot_general(
        agg, w_ref[...], (((1,), (0,)), ((), ())),
        precision=lax.Precision.HIGHEST,
        preferred_element_type=jnp.float32,
    )
    o_ref[...] = x_ref[...] + conv + b_ref[...]


@jax.jit
def kernel(x, edge_index, W, b):
    n, d = x.shape
    e = edge_index.shape[1]
    m_chunks = e // CHUNK

    src3 = edge_index[0].reshape(m_chunks, 1, CHUNK)
    dst3 = edge_index[1].reshape(m_chunks, 1, CHUNK)

    stripe = -(-n // NS)
    stripe += (-stripe) % 8
    zeros = jnp.zeros((stripe, d), jnp.float32)

    partials = _sc_aggregate(n, d, m_chunks)(x, src3, dst3, zeros)

    blk = 1000
    nb = n // blk
    b2 = b.reshape(1, d)
    row_spec = pl.BlockSpec((blk, d), lambda i: (i, 0))
    out = pl.pallas_call(
        _combine_body,
        grid=(nb,),
        in_specs=[
            row_spec,
            pl.BlockSpec((1, blk, d), lambda i: (0, i, 0)),
            pl.BlockSpec((1, blk, d), lambda i: (1, i, 0)),
            pl.BlockSpec((d, d), lambda i: (0, 0)),
            pl.BlockSpec((1, d), lambda i: (0, 0)),
        ],
        out_specs=row_spec,
        out_shape=jax.ShapeDtypeStruct((n, d), jnp.float32),
    )(x, partials, partials, W, b2)
    return out


# speedup vs baseline: 13.6138x; 1.0559x over previous
import functools

import jax
import jax.numpy as jnp
from jax import lax
from jax.experimental import pallas as pl
from jax.experimental.pallas import tpu as pltpu
from jax.experimental.pallas import tpu_sc as plsc

NC = 2
NS = 16
NW = NC * NS
CHUNK = 128
NBUF = 3
IR = 6


def _sc_aggregate(n_rows, d, m_chunks):
    stripe = -(-n_rows // NS)
    stripe += (-stripe) % 8
    tail = n_rows - stripe * (NS - 1)
    assert tail > 0 and tail % 8 == 0 and stripe % 8 == 0
    base_c = m_chunks // NW
    extra = m_chunks - base_c * NW

    mesh = plsc.VectorSubcoreMesh(core_axis_name="c", subcore_axis_name="s")

    @functools.partial(
        pl.kernel,
        out_type=jax.ShapeDtypeStruct((NC, n_rows, d), jnp.float32),
        mesh=mesh,
        scratch_types=[
            pltpu.VMEM((IR, 1, CHUNK), jnp.int32),
            pltpu.VMEM((IR, 1, CHUNK), jnp.int32),
            pltpu.VMEM((CHUNK, d), jnp.float32),
            pltpu.VMEM((CHUNK, d), jnp.float32),
            pltpu.VMEM((CHUNK, d), jnp.float32),
            pltpu.SemaphoreType.DMA,
            pltpu.SemaphoreType.DMA,
            pltpu.SemaphoreType.DMA,
            pltpu.SemaphoreType.DMA,
            pltpu.SemaphoreType.DMA,
            pltpu.SemaphoreType.DMA,
            pltpu.SemaphoreType.DMA,
            pltpu.SemaphoreType.DMA,
            pltpu.SemaphoreType.DMA,
            pltpu.VMEM_SHARED((n_rows, d), jnp.float32),
        ],
    )
    def sc_kernel(x_hbm, src_hbm, dst_hbm, zeros_hbm, out_hbm,
                  src_r, dst_r, b0, b1, b2,
                  i0, i1, i2, i3, i4, i5, g0, g1, g2, acc):
        c = lax.axis_index("c")
        s = lax.axis_index("s")
        wid = s * NC + c
        bufs = (b0, b1, b2)
        gsems = (g0, g1, g2)
        isems = (i0, i1, i2, i3, i4, i5)

        start = base_c * wid + jnp.minimum(wid, extra)
        n_real = base_c + jnp.where(wid < extra, 1, 0)
        t_main = base_c - (base_c % IR)
        ep_max = (base_c % IR) + 1
        row0 = s * stripe

        @pl.when(s < NS - 1)
        def _():
            pltpu.sync_copy(zeros_hbm, acc.at[pl.ds(row0, stripe)])

        @pl.when(s == NS - 1)
        def _():
            pltpu.sync_copy(zeros_hbm.at[pl.ds(0, tail)],
                            acc.at[pl.ds((NS - 1) * stripe, tail)])

        def grow(t):
            return jnp.minimum(start + t, m_chunks - 1)

        def idx_load(t, slot):
            g = grow(t)
            pltpu.make_async_copy(src_hbm.at[g], src_r.at[slot],
                                  isems[slot]).start()
            pltpu.make_async_copy(dst_hbm.at[g], dst_r.at[slot],
                                  isems[slot]).start()

        def idx_wait(slot):
            pltpu.make_async_copy(src_hbm.at[0], src_r.at[slot],
                                  isems[slot]).wait()
            pltpu.make_async_copy(dst_hbm.at[0], dst_r.at[slot],
                                  isems[slot]).wait()

        def gather(islot, ring):
            return pltpu.make_async_copy(
                x_hbm.at[src_r.at[islot, 0]], bufs[ring], gsems[ring])

        for t in range(IR):
            idx_load(t, t)
        for t in range(NBUF):
            idx_wait(t)
            gather(t, t).start()
        plsc.subcore_barrier()

        @pl.loop(0, t_main, step=IR)
        def _(t):
            for r in range(IR):
                tt = t + r
                ring = r % NBUF
                gather(r, ring).wait()

                @pl.when(tt < n_real)
                def _():
                    pltpu.sync_copy(bufs[ring], acc.at[dst_r.at[r, 0]],
                                    add=True)

                @pl.when(tt + IR < n_real)
                def _():
                    idx_load(tt + IR, r)

                nslot = (r + NBUF) % IR

                @pl.when(tt + NBUF < n_real)
                def _():
                    idx_wait(nslot)
                    gather(nslot, ring).start()

        for r_e in range(ep_max):
            tt_e = t_main + r_e
            ring_e = tt_e % NBUF
            islot_e = tt_e % IR

            @pl.when(tt_e < n_real)
            def _():
                gather(islot_e, ring_e).wait()
                pltpu.sync_copy(bufs[ring_e], acc.at[dst_r.at[islot_e, 0]],
                                add=True)

        plsc.subcore_barrier()
        @pl.when(s < NS - 1)
        def _():
            pltpu.sync_copy(acc.at[pl.ds(row0, stripe)],
                            out_hbm.at[c, pl.ds(row0, stripe)])

        @pl.when(s == NS - 1)
        def _():
            pltpu.sync_copy(acc.at[pl.ds((NS - 1) * stripe, tail)],
                            out_hbm.at[c, pl.ds((NS - 1) * stripe, tail)])

    return sc_kernel


def _combine_body(x_ref, p0_ref, p1_ref, w_ref, b_ref, o_ref):
    agg = p0_ref[0] + p1_ref[0]
    conv = lax.dot_general(
        agg, w_ref[...], (((1,), (0,)), ((), ())),
        precision=lax.Precision.HIGHEST,
        preferred_element_type=jnp.float32,
    )
    o_ref[...] = x_ref[...] + conv + b_ref[...]


@jax.jit
def kernel(x, edge_index, W, b):
    n, d = x.shape
    e = edge_index.shape[1]
    m_chunks = e // CHUNK

    src3 = edge_index[0].reshape(m_chunks, 1, CHUNK)
    dst3 = edge_index[1].reshape(m_chunks, 1, CHUNK)

    stripe = -(-n // NS)
    stripe += (-stripe) % 8
    zeros = jnp.zeros((stripe, d), jnp.float32)

    partials = _sc_aggregate(n, d, m_chunks)(x, src3, dst3, zeros)

    blk = 2000
    nb = n // blk
    b2 = b.reshape(1, d)
    row_spec = pl.BlockSpec((blk, d), lambda i: (i, 0))
    out = pl.pallas_call(
        _combine_body,
        grid=(nb,),
        in_specs=[
            row_spec,
            pl.BlockSpec((1, blk, d), lambda i: (0, i, 0)),
            pl.BlockSpec((1, blk, d), lambda i: (1, i, 0)),
            pl.BlockSpec((d, d), lambda i: (0, 0)),
            pl.BlockSpec((1, d), lambda i: (0, 0)),
        ],
        out_specs=row_spec,
        out_shape=jax.ShapeDtypeStruct((n, d), jnp.float32),
    )(x, partials, partials, W, b2)
    return out
